# baseline (speedup 1.0000x reference)
import jax
import jax.numpy as jnp
from jax.experimental import pallas as pl
from jax.experimental.pallas import tpu as pltpu


_TB = 512


def _feat_kernel(x_ref, w1_ref, o_ref):
    o_ref[...] = jnp.dot(
        x_ref[...], w1_ref[...], preferred_element_type=jnp.float32
    ).astype(o_ref.dtype)


def _gcn_kernel(adj_ref, s1_ref, w2_ref, o_ref):
    j = pl.program_id(0)

    a = adj_ref[...].astype(jnp.bfloat16)
    t = jnp.dot(a, s1_ref[...], preferred_element_type=jnp.float32)
    h = jnp.maximum(t, 0.0)
    s2 = jnp.dot(
        h, w2_ref[...], preferred_element_type=jnp.float32
    ).astype(jnp.bfloat16)

    zupd = jax.lax.dot_general(
        a, s2, (((0,), (0,)), ((), ())),
        preferred_element_type=jnp.float32)

    @pl.when(j == 0)
    def _():
        o_ref[...] = zupd

    @pl.when(j > 0)
    def _():
        o_ref[...] += zupd


def kernel(x, adj, gc1_weight, gc2_weight):
    x = x.astype(jnp.float32)
    adj = adj.astype(jnp.float32)
    w1 = gc1_weight.astype(jnp.float32)
    w2 = gc2_weight.astype(jnp.float32)

    n, f = x.shape
    h1 = w1.shape[1]
    h2 = w2.shape[1]
    assert n % _TB == 0, n
    nb = n // _TB

    s1 = pl.pallas_call(
        _feat_kernel,
        out_shape=jax.ShapeDtypeStruct((n, h1), jnp.bfloat16),
        grid=(nb,),
        in_specs=[
            pl.BlockSpec((_TB, f), lambda i: (i, 0)),
            pl.BlockSpec((f, h1), lambda i: (0, 0)),
        ],
        out_specs=pl.BlockSpec((_TB, h1), lambda i: (i, 0)),
        compiler_params=pltpu.CompilerParams(
            dimension_semantics=("arbitrary",)),
    )(x, w1)

    z = pl.pallas_call(
        _gcn_kernel,
        out_shape=jax.ShapeDtypeStruct((n, h2), jnp.float32),
        grid=(nb,),
        in_specs=[
            pl.BlockSpec((_TB, n), lambda j: (j, 0)),
            pl.BlockSpec((n, h1), lambda j: (0, 0)),
            pl.BlockSpec((h1, h2), lambda j: (0, 0)),
        ],
        out_specs=pl.BlockSpec((n, h2), lambda j: (0, 0)),
        compiler_params=pltpu.CompilerParams(
            dimension_semantics=("arbitrary",),
            vmem_limit_bytes=120 * 1024 * 1024,
        ),
    )(adj, s1, w2)

    return z, z, None


# speedup vs baseline: 1.2059x; 1.2059x over previous
import jax
import jax.numpy as jnp
from jax.experimental import pallas as pl
from jax.experimental.pallas import tpu as pltpu


_TI = 512
_TO = 1024


def _feat_kernel(x_ref, w1_ref, o_ref):
    o_ref[...] = jnp.dot(
        x_ref[...], w1_ref[...], preferred_element_type=jnp.float32
    ).astype(o_ref.dtype)


def _gcn_kernel(adj_ref, s1_ref, w2_ref, o_ref, cache_ref, s2_ref):
    p = pl.program_id(0)
    i = pl.program_id(1)

    @pl.when(p == 0)
    def _():
        a = adj_ref[...].astype(jnp.bfloat16)
        cache_ref[pl.ds(i * _TI, _TI), :] = a
        t = jnp.dot(a, s1_ref[...], preferred_element_type=jnp.float32)
        h = jnp.maximum(t, 0.0)
        s2_ref[pl.ds(i * _TI, _TI), :] = jnp.dot(
            h, w2_ref[...], preferred_element_type=jnp.float32
        ).astype(s2_ref.dtype)

    nzb = pl.num_programs(1) * _TI // _TO

    @pl.when((p == 1) & (i < nzb))
    def _():
        a = cache_ref[pl.ds(i * _TO, _TO), :]
        o_ref[pl.ds(i * _TO, _TO), :] = jnp.dot(
            a, s2_ref[...], preferred_element_type=jnp.float32)


def kernel(x, adj, gc1_weight, gc2_weight):
    x = x.astype(jnp.float32)
    adj = adj.astype(jnp.float32)
    w1 = gc1_weight.astype(jnp.float32)
    w2 = gc2_weight.astype(jnp.float32)

    n, f = x.shape
    h1 = w1.shape[1]
    h2 = w2.shape[1]
    assert n % _TI == 0 and n % _TO == 0, n
    nb = n // _TI

    s1 = pl.pallas_call(
        _feat_kernel,
        out_shape=jax.ShapeDtypeStruct((n, h1), jnp.bfloat16),
        grid=(n // 512,),
        in_specs=[
            pl.BlockSpec((512, f), lambda i: (i, 0)),
            pl.BlockSpec((f, h1), lambda i: (0, 0)),
        ],
        out_specs=pl.BlockSpec((512, h1), lambda i: (i, 0)),
        compiler_params=pltpu.CompilerParams(
            dimension_semantics=("arbitrary",)),
    )(x, w1)

    z = pl.pallas_call(
        _gcn_kernel,
        out_shape=jax.ShapeDtypeStruct((n, h2), jnp.float32),
        grid=(2, nb),
        in_specs=[
            pl.BlockSpec((_TI, n),
                         lambda p, i: (jnp.where(p == 0, i, nb - 1), 0)),
            pl.BlockSpec((n, h1), lambda p, i: (0, 0)),
            pl.BlockSpec((h1, h2), lambda p, i: (0, 0)),
        ],
        out_specs=pl.BlockSpec((n, h2), lambda p, i: (0, 0)),
        scratch_shapes=[
            pltpu.VMEM((n, n), jnp.bfloat16),
            pltpu.VMEM((n, h2), jnp.bfloat16),
        ],
        compiler_params=pltpu.CompilerParams(
            dimension_semantics=("arbitrary", "arbitrary"),
            vmem_limit_bytes=120 * 1024 * 1024,
        ),
    )(adj, s1, w2)

    return z, z, None


# speedup vs baseline: 1.2547x; 1.0405x over previous
import jax
import jax.numpy as jnp
from jax.experimental import pallas as pl
from jax.experimental.pallas import tpu as pltpu


_TI = 512
_TO = 1024


def _gcn_kernel(x_ref, adj_ref, w1_ref, w2_ref, o_ref,
                cache_ref, s1_ref, s2_ref):
    p = pl.program_id(0)
    i = pl.program_id(1)

    @pl.when(p == 0)
    def _():
        s1_ref[pl.ds(i * _TI, _TI), :] = jnp.dot(
            x_ref[...], w1_ref[...], preferred_element_type=jnp.float32
        ).astype(s1_ref.dtype)

    @pl.when(p == 1)
    def _():
        a = adj_ref[...].astype(jnp.bfloat16)
        cache_ref[pl.ds(i * _TI, _TI), :] = a
        t = jnp.dot(a, s1_ref[...], preferred_element_type=jnp.float32)
        h = jnp.maximum(t, 0.0)
        s2_ref[pl.ds(i * _TI, _TI), :] = jnp.dot(
            h, w2_ref[...], preferred_element_type=jnp.float32
        ).astype(s2_ref.dtype)

    nzb = pl.num_programs(1) * _TI // _TO

    @pl.when((p == 2) & (i < nzb))
    def _():
        a = cache_ref[pl.ds(i * _TO, _TO), :]
        o_ref[pl.ds(i * _TO, _TO), :] = jnp.dot(
            a, s2_ref[...], preferred_element_type=jnp.float32)


def kernel(x, adj, gc1_weight, gc2_weight):
    x = x.astype(jnp.float32)
    adj = adj.astype(jnp.float32)
    w1 = gc1_weight.astype(jnp.float32)
    w2 = gc2_weight.astype(jnp.float32)

    n, f = x.shape
    h1 = w1.shape[1]
    h2 = w2.shape[1]
    assert n % _TI == 0 and n % _TO == 0, n
    nb = n // _TI

    z = pl.pallas_call(
        _gcn_kernel,
        out_shape=jax.ShapeDtypeStruct((n, h2), jnp.float32),
        grid=(3, nb),
        in_specs=[
            pl.BlockSpec((_TI, f),
                         lambda p, i: (jnp.where(p == 0, i, nb - 1), 0)),
            pl.BlockSpec((_TI, n),
                         lambda p, i: (jnp.where(p == 1, i,
                                                 jnp.where(p == 0, 0,
                                                           nb - 1)), 0)),
            pl.BlockSpec((f, h1), lambda p, i: (0, 0)),
            pl.BlockSpec((h1, h2), lambda p, i: (0, 0)),
        ],
        out_specs=pl.BlockSpec((n, h2), lambda p, i: (0, 0)),
        scratch_shapes=[
            pltpu.VMEM((n, n), jnp.bfloat16),
            pltpu.VMEM((n, h1), jnp.bfloat16),
            pltpu.VMEM((n, h2), jnp.bfloat16),
        ],
        compiler_params=pltpu.CompilerParams(
            dimension_semantics=("arbitrary", "arbitrary"),
            vmem_limit_bytes=120 * 1024 * 1024,
        ),
    )(x, adj, w1, w2)

    return z, z, None


# speedup vs baseline: 1.2555x; 1.0006x over previous
import jax
import jax.numpy as jnp
from jax.experimental import pallas as pl
from jax.experimental.pallas import tpu as pltpu


_TI = 512
_TO = 2048


def _gcn_kernel(x_ref, adj_ref, w1_ref, w2_ref, o_ref,
                cache_ref, s1_ref, s2_ref):
    p = pl.program_id(0)
    i = pl.program_id(1)

    @pl.when(p == 0)
    def _():
        s1_ref[pl.ds(i * _TI, _TI), :] = jnp.dot(
            x_ref[...], w1_ref[...], preferred_element_type=jnp.float32
        ).astype(s1_ref.dtype)

    @pl.when(p == 1)
    def _():
        a = adj_ref[...].astype(jnp.bfloat16)
        cache_ref[pl.ds(i * _TI, _TI), :] = a
        t = jnp.dot(a, s1_ref[...], preferred_element_type=jnp.float32)
        h = jnp.maximum(t, 0.0)
        s2_ref[pl.ds(i * _TI, _TI), :] = jnp.dot(
            h, w2_ref[...], preferred_element_type=jnp.float32
        ).astype(s2_ref.dtype)

    nzb = pl.num_programs(1) * _TI // _TO

    @pl.when((p == 2) & (i < nzb))
    def _():
        a = cache_ref[pl.ds(i * _TO, _TO), :]
        o_ref[pl.ds(i * _TO, _TO), :] = jnp.dot(
            a, s2_ref[...], preferred_element_type=jnp.float32)


def kernel(x, adj, gc1_weight, gc2_weight):
    x = x.astype(jnp.float32)
    adj = adj.astype(jnp.float32)
    w1 = gc1_weight.astype(jnp.float32)
    w2 = gc2_weight.astype(jnp.float32)

    n, f = x.shape
    h1 = w1.shape[1]
    h2 = w2.shape[1]
    assert n % _TI == 0 and n % _TO == 0, n
    nb = n // _TI

    z = pl.pallas_call(
        _gcn_kernel,
        out_shape=jax.ShapeDtypeStruct((n, h2), jnp.float32),
        grid=(3, nb),
        in_specs=[
            pl.BlockSpec((_TI, f),
                         lambda p, i: (jnp.where(p == 0, i, nb - 1), 0)),
            pl.BlockSpec((_TI, n),
                         lambda p, i: (jnp.where(p == 1, i,
                                                 jnp.where(p == 0, 0,
                                                           nb - 1)), 0)),
            pl.BlockSpec((f, h1), lambda p, i: (0, 0)),
            pl.BlockSpec((h1, h2), lambda p, i: (0, 0)),
        ],
        out_specs=pl.BlockSpec((n, h2), lambda p, i: (0, 0)),
        scratch_shapes=[
            pltpu.VMEM((n, n), jnp.bfloat16),
            pltpu.VMEM((n, h1), jnp.bfloat16),
            pltpu.VMEM((n, h2), jnp.bfloat16),
        ],
        compiler_params=pltpu.CompilerParams(
            dimension_semantics=("arbitrary", "arbitrary"),
            vmem_limit_bytes=120 * 1024 * 1024,
        ),
    )(x, adj, w1, w2)

    return z, z, None
